# Initial kernel scaffold; baseline (speedup 1.0000x reference)
import jax
import jax.numpy as jnp
from jax.experimental import pallas as pl


def kernel(x, I, W):
    raise NotImplementedError("write your pallas kernel here")


# speedup vs baseline: 1.5870x; 1.5870x over previous
import functools

import jax
import jax.numpy as jnp
from jax.experimental import pallas as pl
from jax.experimental.pallas import tpu as pltpu

B = 16
C = 32
G = 1024
S = 16
L = 4
M = 32
GB = G * B
GAMMA = 0.01
INV_GAMMA = 100.0


def _combine_body(m_ref, e_ref, w_ref, r_ref, out_ref):
    cs = m_ref[...] + GAMMA * jnp.log(e_ref[...])
    w = w_ref[...]
    wmax = jnp.max(w, axis=1, keepdims=True)
    we = jnp.exp(w - wmax)
    wstar = we / jnp.sum(we, axis=1, keepdims=True)
    denom = jnp.maximum(jnp.max(cs, axis=1, keepdims=True), 1.0)
    wsc = wstar / denom.reshape(1, C)
    h = jnp.dot(wsc, cs, preferred_element_type=jnp.float32)
    mh = jnp.max(h, axis=0, keepdims=True)
    eh = jnp.sum(jnp.exp((h - mh) * INV_GAMMA), axis=0, keepdims=True)
    lse = mh + GAMMA * jnp.log(eh)
    r_new = lse / jnp.maximum(jnp.max(lse), 1.0)
    r_old = r_ref[...]
    mm = jnp.maximum(r_old, r_new)
    pair = jnp.exp((r_old - mm) * INV_GAMMA) + jnp.exp((r_new - mm) * INV_GAMMA)
    l2 = mm + GAMMA * jnp.log(pair)
    out_ref[...] = l2 / jnp.maximum(jnp.max(l2), 1.0)


_combine = pl.pallas_call(
    _combine_body,
    out_shape=jax.ShapeDtypeStruct((1, GB), jnp.float32),
)


def _clause_partials_jnp(xflat, I):
    x_t = xflat.reshape(G, B)
    gath = x_t[I]
    body = jnp.prod(gath, axis=3)
    mb = jnp.max(body, axis=2)
    eb = jnp.sum(jnp.exp((body - mb[:, :, None, :]) * INV_GAMMA), axis=2)
    return mb.reshape(C, GB), eb.reshape(C, GB)


def kernel(x, I, W):
    xflat = x.T.reshape(GB)
    r = xflat.reshape(1, GB)
    for _ in range(2):
        m, e = _clause_partials_jnp(r.reshape(GB), I)
        r = _combine(m, e, W, r)
    return r.reshape(G, B).T


# speedup vs baseline: 25.5581x; 16.1045x over previous
import functools

import jax
import jax.numpy as jnp
from jax import lax
from jax.experimental import pallas as pl
from jax.experimental.pallas import tpu as pltpu
from jax.experimental.pallas import tpu_sc as plsc

B = 16
C = 32
G = 1024
S = 16
L = 4
M = 32
GB = G * B
GAMMA = 0.01
INV_GAMMA = 100.0


def _combine_body(m_ref, e_ref, w_ref, r_ref, out_ref):
    cs = m_ref[...] + GAMMA * jnp.log(e_ref[...])
    w = w_ref[...]
    wmax = jnp.max(w, axis=1, keepdims=True)
    we = jnp.exp(w - wmax)
    wstar = we / jnp.sum(we, axis=1, keepdims=True)
    denom = jnp.maximum(jnp.max(cs, axis=1, keepdims=True), 1.0)
    wsc = wstar / denom.reshape(1, C)
    h = jnp.dot(wsc, cs, preferred_element_type=jnp.float32)
    mh = jnp.max(h, axis=0, keepdims=True)
    eh = jnp.sum(jnp.exp((h - mh) * INV_GAMMA), axis=0, keepdims=True)
    lse = mh + GAMMA * jnp.log(eh)
    r_new = lse / jnp.maximum(jnp.max(lse), 1.0)
    r_old = r_ref[...]
    mm = jnp.maximum(r_old, r_new)
    pair = jnp.exp((r_old - mm) * INV_GAMMA) + jnp.exp((r_new - mm) * INV_GAMMA)
    l2 = mm + GAMMA * jnp.log(pair)
    out_ref[...] = l2 / jnp.maximum(jnp.max(l2), 1.0)


_combine = pl.pallas_call(
    _combine_body,
    out_shape=jax.ShapeDtypeStruct((1, GB), jnp.float32),
)


_sc_mesh = plsc.VectorSubcoreMesh(
    core_axis_name="c", subcore_axis_name="s", num_cores=2, num_subcores=16
)


@functools.partial(
    pl.kernel,
    out_type=[
        jax.ShapeDtypeStruct((C, G, B), jnp.float32),
        jax.ShapeDtypeStruct((C, G, B), jnp.float32),
    ],
    mesh=_sc_mesh,
    compiler_params=pltpu.CompilerParams(
        needs_layout_passes=False, use_tc_tiling_on_sc=False
    ),
    scratch_types=[
        pltpu.VMEM((G * S * L,), jnp.int32),
        pltpu.VMEM((GB,), jnp.float32),
        pltpu.VMEM((S * B,), jnp.float32),
        pltpu.VMEM((G, B), jnp.float32),
        pltpu.VMEM((G, B), jnp.float32),
    ],
)
def _sc_clause(xf_hbm, i_hbm, m_hbm, e_hbm, i_v, xf_v, tr_v, m_v, e_v):
    cid = lax.axis_index("c")
    sid = lax.axis_index("s")
    clause = sid * 2 + cid
    pltpu.sync_copy(xf_hbm, xf_v)
    pltpu.sync_copy(i_hbm.at[clause], i_v)
    iota = lax.iota(jnp.int32, 16)
    iota4 = iota * 4
    tidx = iota * 16

    def g_body(g, carry):
        base = g * (S * L)
        js = [
            plsc.load_gather(i_v, [iota4 + (base + l)]) * B for l in range(L)
        ]
        for b in range(B):
            p = plsc.load_gather(xf_v, [js[0] + b])
            p = p * plsc.load_gather(xf_v, [js[1] + b])
            p = p * plsc.load_gather(xf_v, [js[2] + b])
            p = p * plsc.load_gather(xf_v, [js[3] + b])
            plsc.store_scatter(tr_v, [tidx + b], p)
        rows = [tr_v[pl.ds(s * B, B)] for s in range(S)]
        m = rows[0]
        for s in range(1, S):
            m = jnp.maximum(m, rows[s])
        e = jnp.exp((rows[0] - m) * INV_GAMMA)
        for s in range(1, S):
            e = e + jnp.exp((rows[s] - m) * INV_GAMMA)
        m_v[g] = m
        e_v[g] = e
        return carry

    lax.fori_loop(0, G, g_body, 0)
    pltpu.sync_copy(m_v, m_hbm.at[clause])
    pltpu.sync_copy(e_v, e_hbm.at[clause])


def kernel(x, I, W):
    xflat = x.T.reshape(GB)
    i_flat = I.reshape(C, G * S * L)
    r = xflat.reshape(1, GB)
    for _ in range(2):
        m, e = _sc_clause(r.reshape(GB), i_flat)
        r = _combine(m.reshape(C, GB), e.reshape(C, GB), W, r)
    return r.reshape(G, B).T


# speedup vs baseline: 53.6214x; 2.0980x over previous
import functools

import jax
import jax.numpy as jnp
from jax import lax
from jax.experimental import pallas as pl
from jax.experimental.pallas import tpu as pltpu
from jax.experimental.pallas import tpu_sc as plsc

B = 16
C = 32
G = 1024
S = 16
L = 4
M = 32
GB = G * B
GAMMA = 0.01
INV_GAMMA = 100.0
TRP = 17


def _combine_body(m_ref, e_ref, w_ref, r_ref, out_ref):
    cs = m_ref[...] + GAMMA * jnp.log(e_ref[...])
    w = w_ref[...]
    wmax = jnp.max(w, axis=1, keepdims=True)
    we = jnp.exp(w - wmax)
    wstar = we / jnp.sum(we, axis=1, keepdims=True)
    denom = jnp.maximum(jnp.max(cs, axis=1, keepdims=True), 1.0)
    wsc = wstar / denom.reshape(1, C)
    h = jnp.dot(wsc, cs, preferred_element_type=jnp.float32)
    mh = jnp.max(h, axis=0, keepdims=True)
    eh = jnp.sum(jnp.exp((h - mh) * INV_GAMMA), axis=0, keepdims=True)
    lse = mh + GAMMA * jnp.log(eh)
    r_new = lse / jnp.maximum(jnp.max(lse), 1.0)
    r_old = r_ref[...]
    mm = jnp.maximum(r_old, r_new)
    pair = jnp.exp((r_old - mm) * INV_GAMMA) + jnp.exp((r_new - mm) * INV_GAMMA)
    l2 = mm + GAMMA * jnp.log(pair)
    out_ref[...] = l2 / jnp.maximum(jnp.max(l2), 1.0)


_combine = pl.pallas_call(
    _combine_body,
    out_shape=jax.ShapeDtypeStruct((1, GB), jnp.float32),
)


_sc_mesh = plsc.VectorSubcoreMesh(
    core_axis_name="c", subcore_axis_name="s", num_cores=2, num_subcores=16
)


@functools.partial(
    pl.kernel,
    out_type=[
        jax.ShapeDtypeStruct((C, G, B), jnp.float32),
        jax.ShapeDtypeStruct((C, G, B), jnp.float32),
    ],
    mesh=_sc_mesh,
    compiler_params=pltpu.CompilerParams(
        needs_layout_passes=False, use_tc_tiling_on_sc=False
    ),
    scratch_types=[
        pltpu.VMEM((G * S * L,), jnp.int32),
        pltpu.VMEM((GB,), jnp.float32),
        pltpu.VMEM((16, S * TRP), jnp.float32),
        pltpu.VMEM((G, B), jnp.float32),
        pltpu.VMEM((G, B), jnp.float32),
    ],
)
def _sc_clause(xf_hbm, i_hbm, m_hbm, e_hbm, i_v, xf_v, tr_v, m_v, e_v):
    cid = lax.axis_index("c")
    sid = lax.axis_index("s")
    clause = sid * 2 + cid
    pltpu.sync_copy(xf_hbm, xf_v)
    pltpu.sync_copy(i_hbm.at[clause], i_v)
    iota = lax.iota(jnp.int32, 16)
    iota4 = iota * 4
    iota_tr = iota * TRP

    def outer(go, carry):
        @plsc.parallel_loop(0, 16, unroll=1)
        def inner(gi):
            g = go * 16 + gi
            tr = tr_v.at[gi]
            base = g * (S * L)
            js = [
                plsc.load_gather(i_v, [iota4 + (base + l)]) for l in range(L)
            ]
            ps = []
            for b in range(B):
                off = b * G
                p01 = plsc.load_gather(xf_v, [js[0] + off]) * plsc.load_gather(
                    xf_v, [js[1] + off]
                )
                p23 = plsc.load_gather(xf_v, [js[2] + off]) * plsc.load_gather(
                    xf_v, [js[3] + off]
                )
                ps.append(p01 * p23)
            for b in range(B):
                plsc.store_scatter(tr, [iota_tr + b], ps[b])
            rows = [
                plsc.load_gather(tr, [iota + (s * TRP)]) for s in range(S)
            ]
            t = rows
            while len(t) > 1:
                t = [
                    jnp.maximum(t[2 * i], t[2 * i + 1])
                    for i in range(len(t) // 2)
                ]
            m = t[0]
            es = [jnp.exp((r - m) * INV_GAMMA) for r in rows]
            while len(es) > 1:
                es = [es[2 * i] + es[2 * i + 1] for i in range(len(es) // 2)]
            m_v[g] = m
            e_v[g] = es[0]

        return carry

    lax.fori_loop(0, G // 16, outer, 0)
    pltpu.sync_copy(m_v, m_hbm.at[clause])
    pltpu.sync_copy(e_v, e_hbm.at[clause])


def kernel(x, I, W):
    xf = x.reshape(GB)
    i_flat = I.reshape(C, G * S * L)
    r_gb = x.T.reshape(1, GB)
    for _ in range(2):
        m, e = _sc_clause(xf, i_flat)
        r_gb = _combine(m.reshape(C, GB), e.reshape(C, GB), W, r_gb)
        xf = r_gb.reshape(G, B).T.reshape(GB)
    return xf.reshape(B, G)


# speedup vs baseline: 115.8808x; 2.1611x over previous
import functools

import jax
import jax.numpy as jnp
from jax import lax
from jax.experimental import pallas as pl
from jax.experimental.pallas import tpu as pltpu
from jax.experimental.pallas import tpu_sc as plsc

B = 16
C = 32
G = 1024
S = 16
L = 4
M = 32
GB = G * B
GAMMA = 0.01
INV_GAMMA = 100.0
TRP = 17


def _combine_body(m_ref, e_ref, w_ref, r_ref, out_ref):
    cs = m_ref[...] + GAMMA * jnp.log(e_ref[...])
    cs3 = cs.reshape(C, GB // 128, 128)
    w = w_ref[...]
    wmax = jnp.max(w, axis=1, keepdims=True)
    we = jnp.exp(w - wmax)
    wstar = we / jnp.sum(we, axis=1, keepdims=True)
    denom = jnp.maximum(jnp.max(cs3, axis=(1, 2)), 1.0)
    wsc = wstar / denom.reshape(1, C)
    h = lax.dot_general(
        wsc,
        cs3,
        (((1,), (0,)), ((), ())),
        preferred_element_type=jnp.float32,
    )
    mh = jnp.max(h, axis=0, keepdims=True)
    eh = jnp.sum(jnp.exp((h - mh) * INV_GAMMA), axis=0, keepdims=True)
    lse = (mh + GAMMA * jnp.log(eh)).reshape(GB // 128, 128)
    r_new = lse / jnp.maximum(jnp.max(lse), 1.0)
    r_old = r_ref[...]
    mm = jnp.maximum(r_old, r_new)
    pair = jnp.exp((r_old - mm) * INV_GAMMA) + jnp.exp((r_new - mm) * INV_GAMMA)
    l2 = mm + GAMMA * jnp.log(pair)
    out_ref[...] = l2 / jnp.maximum(jnp.max(l2), 1.0)


_combine = pl.pallas_call(
    _combine_body,
    out_shape=jax.ShapeDtypeStruct((GB // 128, 128), jnp.float32),
)


_sc_mesh = plsc.VectorSubcoreMesh(
    core_axis_name="c", subcore_axis_name="s", num_cores=2, num_subcores=16
)


G1 = G + 1


@functools.partial(
    pl.kernel,
    out_type=[
        jax.ShapeDtypeStruct((C, B, G), jnp.float32),
        jax.ShapeDtypeStruct((C, B, G), jnp.float32),
    ],
    mesh=_sc_mesh,
    compiler_params=pltpu.CompilerParams(
        needs_layout_passes=False, use_tc_tiling_on_sc=False
    ),
    scratch_types=[
        pltpu.VMEM((G * S * L // 128, 128), jnp.int32),
        pltpu.VMEM((GB,), jnp.float32),
        pltpu.VMEM((32, S * TRP), jnp.float32),
        pltpu.VMEM((B, G1), jnp.float32),
        pltpu.VMEM((B, G1), jnp.float32),
    ],
)
def _sc_clause(xf_hbm, i_hbm, m_hbm, e_hbm, i_v, xf_v, tr_v, m_v, e_v):
    cid = lax.axis_index("c")
    sid = lax.axis_index("s")
    clause = sid * 2 + cid
    pltpu.sync_copy(xf_hbm, xf_v)
    pltpu.sync_copy(i_hbm.at[clause], i_v)
    iota = lax.iota(jnp.int32, 16)
    iota4 = iota * 4
    iota_tr = iota * TRP

    def outer(go, carry):
        @plsc.parallel_loop(0, 32, unroll=1)
        def gather_phase(gi):
            g = go * 32 + gi
            tr = tr_v.at[gi]
            base = g * (S * L)
            js = [
                plsc.load_gather(
                    i_v,
                    [
                        (iota4 + (base + l)) >> 7,
                        (iota4 + (base + l)) & 127,
                    ],
                )
                for l in range(L)
            ]
            for b in range(B):
                off = b * G
                p01 = plsc.load_gather(xf_v, [js[0] + off]) * plsc.load_gather(
                    xf_v, [js[1] + off]
                )
                p23 = plsc.load_gather(xf_v, [js[2] + off]) * plsc.load_gather(
                    xf_v, [js[3] + off]
                )
                plsc.store_scatter(tr, [iota_tr + b], p01 * p23)

        @plsc.parallel_loop(0, 32, unroll=1)
        def reduce_phase(gi):
            g = go * 32 + gi
            tr = tr_v.at[gi]
            rows = [plsc.load_gather(tr, [iota + (s * TRP)]) for s in range(S)]
            t = rows
            while len(t) > 1:
                t = [
                    jnp.maximum(t[2 * i], t[2 * i + 1])
                    for i in range(len(t) // 2)
                ]
            m = t[0]
            es = [jnp.exp((r - m) * INV_GAMMA) for r in rows]
            while len(es) > 1:
                es = [es[2 * i] + es[2 * i + 1] for i in range(len(es) // 2)]
            g_vec = jnp.broadcast_to(g, (16,))
            plsc.store_scatter(m_v, [iota, g_vec], m)
            plsc.store_scatter(e_v, [iota, g_vec], es[0])

        return carry

    lax.fori_loop(0, G // 32, outer, 0)
    pltpu.sync_copy(m_v.at[:, pl.ds(0, G)], m_hbm.at[clause])
    pltpu.sync_copy(e_v.at[:, pl.ds(0, G)], e_hbm.at[clause])


def kernel(x, I, W):
    xf = x.reshape(GB)
    i_flat = I.reshape(C, G * S * L // 128, 128)
    r = xf.reshape(GB // 128, 128)
    for _ in range(2):
        m, e = _sc_clause(xf, i_flat)
        r = _combine(
            m.reshape(C * GB // 128, 128), e.reshape(C * GB // 128, 128), W, r
        )
        xf = r.reshape(GB)
    return r.reshape(B, G)
